# baseline (speedup 1.0000x reference)
import functools

import jax
import jax.numpy as jnp
from jax import lax
from jax.experimental import pallas as pl
from jax.experimental.pallas import tpu as pltpu
from jax.experimental.pallas import tpu_sc as plsc

_SHARD_SIZE = 2048
_NUM_SHARDS = 49
_EMBED_DIM = 64
_BATCH = 4096
_SEQ_LEN = 200

_TOT = _BATCH * _SEQ_LEN
_NW = 32
_PER_W = _TOT // _NW
_ROWS_W = _BATCH // _NW
_HALF = _SEQ_LEN // 2
_CHUNK = 104
_NCH = 2 * _ROWS_W
_NBUF = 8
_G = 4
_NOUT = _NCH // _NBUF

_mesh = plsc.VectorSubcoreMesh(core_axis_name="c", subcore_axis_name="s")


@functools.partial(
    pl.kernel,
    out_type=jax.ShapeDtypeStruct((_BATCH, _SEQ_LEN, _EMBED_DIM), jnp.float32),
    mesh=_mesh,
    compiler_params=pltpu.CompilerParams(use_tc_tiling_on_sc=False),
    scratch_types=[
        pltpu.VMEM((_NCH, _CHUNK), jnp.int32),
        pltpu.VMEM((_NBUF, _CHUNK, _EMBED_DIM), jnp.float32),
        [pltpu.SemaphoreType.DMA] * _NBUF,
        [pltpu.SemaphoreType.DMA] * _NBUF,
    ],
)
def _gather_kernel(idx_hbm, table_hbm, out_hbm, idx_v, rows, gsems, ssems):
    wid = lax.axis_index("s") * 2 + lax.axis_index("c")
    row_base = wid * _NCH
    brow_base = wid * _ROWS_W

    pltpu.sync_copy(idx_hbm.at[pl.ds(row_base, _NCH)], idx_v)

    def start_gather(j, b):
        pltpu.async_copy(table_hbm.at[idx_v.at[j]], rows.at[b], gsems[b])

    def wait_gather(j, b):
        pltpu.make_async_copy(
            table_hbm.at[idx_v.at[j]], rows.at[b], gsems[b]
        ).wait()

    def out_slice(j):
        return out_hbm.at[
            brow_base + lax.div(j, 2), pl.ds(lax.rem(j, 2) * _HALF, _HALF)
        ]

    def start_store(j, b):
        pltpu.async_copy(rows.at[b, pl.ds(0, _HALF)], out_slice(j), ssems[b])

    def wait_store(j, b):
        pltpu.make_async_copy(
            rows.at[b, pl.ds(0, _HALF)], out_slice(j), ssems[b]
        ).wait()

    for j in range(_G):
        start_gather(j, j)

    for b in range(_NBUF):
        wait_gather(b, b)
        start_store(b, b)
        bg = (b + _G) % _NBUF
        if b >= _G:
            wait_store(b - _G, bg)
        start_gather(b + _G, bg)

    def body(k, _):
        i0 = k * _NBUF
        for b in range(_NBUF):
            i = i0 + b
            wait_gather(i, b)
            start_store(i, b)
            bg = (b + _G) % _NBUF
            wait_store(i - _G, bg)
            start_gather(i + _G, bg)
        return ()

    lax.fori_loop(1, _NOUT - 1, body, ())

    i0 = (_NOUT - 1) * _NBUF
    for b in range(_NBUF):
        i = i0 + b
        wait_gather(i, b)
        start_store(i, b)
        bg = (b + _G) % _NBUF
        if b < _NBUF - _G:
            wait_store(i - _G, bg)
            start_gather(i + _G, bg)

    for b in range(_NBUF):
        wait_store(i0 + b, b)


def kernel(token_ids, shard_weights):
    idx = token_ids.reshape(_BATCH * 2, _HALF)
    idx = jnp.pad(idx, ((0, 0), (0, _CHUNK - _HALF)))
    table = shard_weights.reshape(_NUM_SHARDS * _SHARD_SIZE, _EMBED_DIM)
    return _gather_kernel(idx, table)


# speedup vs baseline: 1.2269x; 1.2269x over previous
import functools

import jax
import jax.numpy as jnp
from jax import lax
from jax.experimental import pallas as pl
from jax.experimental.pallas import tpu as pltpu
from jax.experimental.pallas import tpu_sc as plsc

_SHARD_SIZE = 2048
_NUM_SHARDS = 49
_EMBED_DIM = 64
_BATCH = 4096
_SEQ_LEN = 200

_NW = 32
_CHUNK = 128
_NTC = _BATCH // _CHUNK
_NCH = _SEQ_LEN
_NBUF = 4
_G = 2
_NOUT = _NCH // _NBUF

_mesh = plsc.VectorSubcoreMesh(core_axis_name="c", subcore_axis_name="s")


@functools.partial(
    pl.kernel,
    out_type=jax.ShapeDtypeStruct(
        (_SEQ_LEN, _EMBED_DIM // 8, _NTC, 8, _CHUNK), jnp.float32
    ),
    mesh=_mesh,
    compiler_params=pltpu.CompilerParams(
        use_tc_tiling_on_sc=False, needs_layout_passes=False
    ),
    scratch_types=[
        pltpu.VMEM((_SEQ_LEN, _CHUNK), jnp.int32),
        pltpu.VMEM((_NBUF, _CHUNK, _EMBED_DIM), jnp.float32),
        pltpu.VMEM((_NBUF, _EMBED_DIM // 8, 1, 8, _CHUNK), jnp.float32),
        [pltpu.SemaphoreType.DMA] * _NBUF,
        [pltpu.SemaphoreType.DMA] * _NBUF,
    ],
)
def _gather_kernel(ids_hbm, table_hbm, out_hbm, idx_v, rows, tbuf, gsems, ssems):
    wid = lax.axis_index("s") * 2 + lax.axis_index("c")

    pltpu.sync_copy(ids_hbm.at[:, pl.ds(wid * _CHUNK, _CHUNK)], idx_v)

    riota = lax.iota(jnp.int32, 16)
    rks = [riota + 16 * k for k in range(8)]

    def start_gather(j, b):
        pltpu.async_copy(table_hbm.at[idx_v.at[j]], rows.at[b], gsems[b])

    def wait_gather(j, b):
        pltpu.make_async_copy(
            table_hbm.at[idx_v.at[j]], rows.at[b], gsems[b]
        ).wait()

    def transpose(b):
        def d_body(d, _):
            tr = lax.div(d, 8)
            r = lax.rem(d, 8)
            cols = jnp.full((16,), d, jnp.int32)
            for k in range(8):
                v = plsc.load_gather(rows.at[b], [rks[k], cols])
                tbuf[b, tr, 0, r, pl.ds(16 * k, 16)] = v
            return ()

        lax.fori_loop(0, _EMBED_DIM, d_body, ())

    def start_store(j, b):
        pltpu.async_copy(
            tbuf.at[b], out_hbm.at[j, :, pl.ds(wid, 1)], ssems[b]
        )

    def wait_store(j, b):
        pltpu.make_async_copy(
            tbuf.at[b], out_hbm.at[j, :, pl.ds(wid, 1)], ssems[b]
        ).wait()

    for j in range(_G):
        start_gather(j, j)

    for b in range(_NBUF):
        wait_gather(b, b)
        start_gather(b + _G, (b + _G) % _NBUF)
        transpose(b)
        start_store(b, b)

    def body(k, _):
        i0 = k * _NBUF
        for b in range(_NBUF):
            i = i0 + b
            wait_gather(i, b)
            wait_store(i - _NBUF, b)
            start_gather(i + _G, (b + _G) % _NBUF)
            transpose(b)
            start_store(i, b)
        return ()

    lax.fori_loop(1, _NOUT - 1, body, ())

    i0 = (_NOUT - 1) * _NBUF
    for b in range(_NBUF):
        i = i0 + b
        wait_gather(i, b)
        wait_store(i - _NBUF, b)
        if b < _NBUF - _G:
            start_gather(i + _G, (b + _G) % _NBUF)
        transpose(b)
        start_store(i, b)

    for b in range(_NBUF):
        wait_store(i0 + b, b)


def kernel(token_ids, shard_weights):
    ids_t = token_ids.T
    table = shard_weights.reshape(_NUM_SHARDS * _SHARD_SIZE, _EMBED_DIM)
    out5 = _gather_kernel(ids_t, table)
    out_phys = jnp.transpose(out5, (0, 1, 3, 2, 4)).reshape(
        _SEQ_LEN, _EMBED_DIM, _BATCH
    )
    return jnp.transpose(out_phys, (2, 0, 1))


# speedup vs baseline: 2.0995x; 1.7113x over previous
import functools

import jax
import jax.numpy as jnp
from jax import lax
from jax.experimental import pallas as pl
from jax.experimental.pallas import tpu as pltpu
from jax.experimental.pallas import tpu_sc as plsc

_SHARD_SIZE = 2048
_NUM_SHARDS = 49
_EMBED_DIM = 64
_BATCH = 4096
_SEQ_LEN = 200

_NW = 32
_CHUNK = 128
_NTC = _BATCH // _CHUNK
_NCH = _SEQ_LEN
_NBUF = 4
_G = 2
_NOUT = _NCH // _NBUF

_mesh = plsc.VectorSubcoreMesh(core_axis_name="c", subcore_axis_name="s")


@functools.partial(
    pl.kernel,
    out_type=jax.ShapeDtypeStruct(
        (_SEQ_LEN, _EMBED_DIM // 8, _NTC, 8, _CHUNK), jnp.float32
    ),
    mesh=_mesh,
    compiler_params=pltpu.CompilerParams(
        use_tc_tiling_on_sc=False, needs_layout_passes=False
    ),
    scratch_types=[
        pltpu.VMEM((_SEQ_LEN, _CHUNK), jnp.int32),
        pltpu.VMEM((_NBUF, _CHUNK, _EMBED_DIM), jnp.float32),
        pltpu.VMEM((_NBUF, _EMBED_DIM // 8, 1, 8, _CHUNK), jnp.float32),
        [pltpu.SemaphoreType.DMA] * _NBUF,
        [pltpu.SemaphoreType.DMA] * _NBUF,
    ],
)
def _gather_kernel(ids_hbm, table_hbm, out_hbm, idx_v, rows, tbuf, gsems, ssems):
    wid = lax.axis_index("s") * 2 + lax.axis_index("c")

    pltpu.sync_copy(ids_hbm.at[:, pl.ds(wid * _CHUNK, _CHUNK)], idx_v)

    riota = lax.iota(jnp.int32, 16)
    rks = [riota + 16 * k for k in range(8)]

    def start_gather(j, b):
        pltpu.async_copy(table_hbm.at[idx_v.at[j]], rows.at[b], gsems[b])

    def wait_gather(j, b):
        pltpu.make_async_copy(
            table_hbm.at[idx_v.at[j]], rows.at[b], gsems[b]
        ).wait()

    def transpose(b):
        @plsc.parallel_loop(0, _EMBED_DIM, unroll=8)
        def _(d):
            tr = lax.div(d, 8)
            r = lax.rem(d, 8)
            cols = jnp.full((16,), d, jnp.int32)
            for k in range(8):
                v = plsc.load_gather(rows.at[b], [rks[k], cols])
                tbuf[b, tr, 0, r, pl.ds(16 * k, 16)] = v

    def start_store(j, b):
        pltpu.async_copy(
            tbuf.at[b], out_hbm.at[j, :, pl.ds(wid, 1)], ssems[b]
        )

    def wait_store(j, b):
        pltpu.make_async_copy(
            tbuf.at[b], out_hbm.at[j, :, pl.ds(wid, 1)], ssems[b]
        ).wait()

    for j in range(_G):
        start_gather(j, j)

    for b in range(_NBUF):
        wait_gather(b, b)
        start_gather(b + _G, (b + _G) % _NBUF)
        transpose(b)
        start_store(b, b)

    def body(k, _):
        i0 = k * _NBUF
        for b in range(_NBUF):
            i = i0 + b
            wait_gather(i, b)
            wait_store(i - _NBUF, b)
            start_gather(i + _G, (b + _G) % _NBUF)
            transpose(b)
            start_store(i, b)
        return ()

    lax.fori_loop(1, _NOUT - 1, body, ())

    i0 = (_NOUT - 1) * _NBUF
    for b in range(_NBUF):
        i = i0 + b
        wait_gather(i, b)
        wait_store(i - _NBUF, b)
        if b < _NBUF - _G:
            start_gather(i + _G, (b + _G) % _NBUF)
        transpose(b)
        start_store(i, b)

    for b in range(_NBUF):
        wait_store(i0 + b, b)


def kernel(token_ids, shard_weights):
    ids_t = token_ids.T
    table = shard_weights.reshape(_NUM_SHARDS * _SHARD_SIZE, _EMBED_DIM)
    out5 = _gather_kernel(ids_t, table)
    out_phys = jnp.transpose(out5, (0, 1, 3, 2, 4)).reshape(
        _SEQ_LEN, _EMBED_DIM, _BATCH
    )
    return jnp.transpose(out_phys, (2, 0, 1))


# speedup vs baseline: 6.7689x; 3.2240x over previous
import functools

import jax
import jax.numpy as jnp
from jax import lax
from jax.experimental import pallas as pl
from jax.experimental.pallas import tpu as pltpu
from jax.experimental.pallas import tpu_sc as plsc

_SHARD_SIZE = 2048
_NUM_SHARDS = 49
_EMBED_DIM = 64
_BATCH = 4096
_SEQ_LEN = 200

_NW = 32
_CHUNK = 128
_NTC = _BATCH // _CHUNK
_NCH = _SEQ_LEN
_NBUF = 4
_G = 2
_NOUT = _NCH // _NBUF

_mesh = plsc.VectorSubcoreMesh(core_axis_name="c", subcore_axis_name="s")


@functools.partial(
    pl.kernel,
    out_type=jax.ShapeDtypeStruct(
        (_SEQ_LEN, _EMBED_DIM // 8, _NTC, 8, _CHUNK), jnp.float32
    ),
    mesh=_mesh,
    compiler_params=pltpu.CompilerParams(
        use_tc_tiling_on_sc=False, needs_layout_passes=False
    ),
    scratch_types=[
        pltpu.VMEM((_SEQ_LEN, _CHUNK), jnp.int32),
        pltpu.VMEM((_NBUF, _CHUNK, _EMBED_DIM), jnp.float32),
        pltpu.VMEM((_NBUF, _EMBED_DIM // 8, 1, 8, _CHUNK), jnp.float32),
        [pltpu.SemaphoreType.DMA] * _NBUF,
        [pltpu.SemaphoreType.DMA] * _NBUF,
    ],
)
def _gather_kernel(ids_hbm, table_hbm, out_hbm, idx_v, rows, tbuf, gsems, ssems):
    wid = lax.axis_index("s") * 2 + lax.axis_index("c")

    pltpu.sync_copy(ids_hbm.at[:, pl.ds(wid * _CHUNK, _CHUNK)], idx_v)

    riota = lax.iota(jnp.int32, 16)
    cvecs = [riota + 16 * k for k in range(8)]
    zvec = jnp.zeros((16,), jnp.int32)

    def start_gather(j, b):
        pltpu.async_copy(table_hbm.at[idx_v.at[j]], rows.at[b], gsems[b])

    def wait_gather(j, b):
        pltpu.make_async_copy(
            table_hbm.at[idx_v.at[j]], rows.at[b], gsems[b]
        ).wait()

    def transpose(b):
        @plsc.parallel_loop(0, _EMBED_DIM, unroll=8)
        def _(d):
            dvec = (riota + d) & 63
            trv = dvec >> 3
            rv = dvec & 7
            for k in range(8):
                v = plsc.load_gather(rows.at[b], [cvecs[k], dvec])
                plsc.store_scatter(
                    tbuf.at[b], [trv, zvec, rv, cvecs[k]], v
                )

    def start_store(j, b):
        pltpu.async_copy(
            tbuf.at[b], out_hbm.at[j, :, pl.ds(wid, 1)], ssems[b]
        )

    def wait_store(j, b):
        pltpu.make_async_copy(
            tbuf.at[b], out_hbm.at[j, :, pl.ds(wid, 1)], ssems[b]
        ).wait()

    for j in range(_G):
        start_gather(j, j)

    for b in range(_NBUF):
        wait_gather(b, b)
        start_gather(b + _G, (b + _G) % _NBUF)
        transpose(b)
        start_store(b, b)

    def body(k, _):
        i0 = k * _NBUF
        for b in range(_NBUF):
            i = i0 + b
            wait_gather(i, b)
            wait_store(i - _NBUF, b)
            start_gather(i + _G, (b + _G) % _NBUF)
            transpose(b)
            start_store(i, b)
        return ()

    lax.fori_loop(1, _NOUT - 1, body, ())

    i0 = (_NOUT - 1) * _NBUF
    for b in range(_NBUF):
        i = i0 + b
        wait_gather(i, b)
        wait_store(i - _NBUF, b)
        if b < _NBUF - _G:
            start_gather(i + _G, (b + _G) % _NBUF)
        transpose(b)
        start_store(i, b)

    for b in range(_NBUF):
        wait_store(i0 + b, b)


def kernel(token_ids, shard_weights):
    ids_t = token_ids.T
    table = shard_weights.reshape(_NUM_SHARDS * _SHARD_SIZE, _EMBED_DIM)
    out5 = _gather_kernel(ids_t, table)
    out_phys = jnp.transpose(out5, (0, 1, 3, 2, 4)).reshape(
        _SEQ_LEN, _EMBED_DIM, _BATCH
    )
    return jnp.transpose(out_phys, (2, 0, 1))


# speedup vs baseline: 6.9823x; 1.0315x over previous
import functools

import jax
import jax.numpy as jnp
from jax import lax
from jax.experimental import pallas as pl
from jax.experimental.pallas import tpu as pltpu
from jax.experimental.pallas import tpu_sc as plsc

_SHARD_SIZE = 2048
_NUM_SHARDS = 49
_EMBED_DIM = 64
_BATCH = 4096
_SEQ_LEN = 200

_NW = 32
_CHUNK = 128
_NTC = _BATCH // _CHUNK
_NCH = _SEQ_LEN
_NBUF = 5
_G = 3
_NOUT = _NCH // _NBUF

_mesh = plsc.VectorSubcoreMesh(core_axis_name="c", subcore_axis_name="s")


@functools.partial(
    pl.kernel,
    out_type=jax.ShapeDtypeStruct(
        (_SEQ_LEN, _EMBED_DIM // 8, _NTC, 8, _CHUNK), jnp.float32
    ),
    mesh=_mesh,
    compiler_params=pltpu.CompilerParams(
        use_tc_tiling_on_sc=False, needs_layout_passes=False
    ),
    scratch_types=[
        pltpu.VMEM((_SEQ_LEN, _CHUNK), jnp.int32),
        pltpu.VMEM((_NBUF, _CHUNK, _EMBED_DIM), jnp.float32),
        pltpu.VMEM((_NBUF, _EMBED_DIM // 8, 1, 8, _CHUNK), jnp.float32),
        [pltpu.SemaphoreType.DMA] * _NBUF,
        [pltpu.SemaphoreType.DMA] * _NBUF,
    ],
)
def _gather_kernel(ids_hbm, table_hbm, out_hbm, idx_v, rows, tbuf, gsems, ssems):
    wid = lax.axis_index("s") * 2 + lax.axis_index("c")

    pltpu.sync_copy(ids_hbm.at[:, pl.ds(wid * _CHUNK, _CHUNK)], idx_v)

    riota = lax.iota(jnp.int32, 16)
    cvecs = [riota + 16 * k for k in range(8)]
    zvec = jnp.zeros((16,), jnp.int32)

    def start_gather(j, b):
        pltpu.async_copy(table_hbm.at[idx_v.at[j]], rows.at[b], gsems[b])

    def wait_gather(j, b):
        pltpu.make_async_copy(
            table_hbm.at[idx_v.at[j]], rows.at[b], gsems[b]
        ).wait()

    def transpose(b):
        @plsc.parallel_loop(0, _EMBED_DIM, unroll=8)
        def _(d):
            dvec = (riota + d) & 63
            trv = dvec >> 3
            rv = dvec & 7
            for k in range(8):
                v = plsc.load_gather(rows.at[b], [cvecs[k], dvec])
                plsc.store_scatter(
                    tbuf.at[b], [trv, zvec, rv, cvecs[k]], v
                )

    def start_store(j, b):
        pltpu.async_copy(
            tbuf.at[b], out_hbm.at[j, :, pl.ds(wid, 1)], ssems[b]
        )

    def wait_store(j, b):
        pltpu.make_async_copy(
            tbuf.at[b], out_hbm.at[j, :, pl.ds(wid, 1)], ssems[b]
        ).wait()

    for j in range(_G):
        start_gather(j, j)

    for b in range(_NBUF):
        wait_gather(b, b)
        start_gather(b + _G, (b + _G) % _NBUF)
        transpose(b)
        start_store(b, b)

    def body(k, _):
        i0 = k * _NBUF
        for b in range(_NBUF):
            i = i0 + b
            wait_gather(i, b)
            wait_store(i - _NBUF, b)
            start_gather(i + _G, (b + _G) % _NBUF)
            transpose(b)
            start_store(i, b)
        return ()

    lax.fori_loop(1, _NOUT - 1, body, ())

    i0 = (_NOUT - 1) * _NBUF
    for b in range(_NBUF):
        i = i0 + b
        wait_gather(i, b)
        wait_store(i - _NBUF, b)
        if b < _NBUF - _G:
            start_gather(i + _G, (b + _G) % _NBUF)
        transpose(b)
        start_store(i, b)

    for b in range(_NBUF):
        wait_store(i0 + b, b)


def kernel(token_ids, shard_weights):
    ids_t = token_ids.T
    table = shard_weights.reshape(_NUM_SHARDS * _SHARD_SIZE, _EMBED_DIM)
    out5 = _gather_kernel(ids_t, table)
    out_phys = jnp.transpose(out5, (0, 1, 3, 2, 4)).reshape(
        _SEQ_LEN, _EMBED_DIM, _BATCH
    )
    return jnp.transpose(out_phys, (2, 0, 1))


# speedup vs baseline: 7.0798x; 1.0140x over previous
import functools

import jax
import jax.numpy as jnp
from jax import lax
from jax.experimental import pallas as pl
from jax.experimental.pallas import tpu as pltpu
from jax.experimental.pallas import tpu_sc as plsc

_SHARD_SIZE = 2048
_NUM_SHARDS = 49
_EMBED_DIM = 64
_BATCH = 4096
_SEQ_LEN = 200

_NW = 32
_CHUNK = 128
_NTC = _BATCH // _CHUNK
_NCH = _SEQ_LEN
_NBUF = 5
_G = 3
_NOUT = _NCH // _NBUF

_mesh = plsc.VectorSubcoreMesh(core_axis_name="c", subcore_axis_name="s")


@functools.partial(
    pl.kernel,
    out_type=jax.ShapeDtypeStruct(
        (_SEQ_LEN, _EMBED_DIM // 8, _NTC, 8, _CHUNK), jnp.float32
    ),
    mesh=_mesh,
    compiler_params=pltpu.CompilerParams(
        use_tc_tiling_on_sc=False, needs_layout_passes=False
    ),
    scratch_types=[
        pltpu.VMEM((_SEQ_LEN, _CHUNK), jnp.int32),
        pltpu.VMEM((_NBUF, _CHUNK, _EMBED_DIM), jnp.float32),
        pltpu.VMEM((_NBUF, _EMBED_DIM // 8, 1, 8, _CHUNK), jnp.float32),
        [pltpu.SemaphoreType.DMA] * _NBUF,
        [pltpu.SemaphoreType.DMA] * _NBUF,
    ],
)
def _gather_kernel(ids_hbm, table_hbm, out_hbm, idx_v, rows, tbuf, gsems, ssems):
    wid = lax.axis_index("s") * 2 + lax.axis_index("c")

    pltpu.sync_copy(ids_hbm.at[:, pl.ds(wid * _CHUNK, _CHUNK)], idx_v)

    riota = lax.iota(jnp.int32, 16)
    cvecs = [riota + 16 * k for k in range(8)]
    zvec = jnp.zeros((16,), jnp.int32)

    def start_gather(j, b):
        pltpu.async_copy(table_hbm.at[idx_v.at[j]], rows.at[b], gsems[b])

    def wait_gather(j, b):
        pltpu.make_async_copy(
            table_hbm.at[idx_v.at[j]], rows.at[b], gsems[b]
        ).wait()

    def transpose(b):
        @plsc.parallel_loop(0, _EMBED_DIM, unroll=16)
        def _(d):
            dvec = (riota + d) & 63
            trv = dvec >> 3
            rv = dvec & 7
            for k in range(8):
                v = plsc.load_gather(rows.at[b], [cvecs[k], dvec])
                plsc.store_scatter(
                    tbuf.at[b], [trv, zvec, rv, cvecs[k]], v
                )

    def start_store(j, b):
        pltpu.async_copy(
            tbuf.at[b], out_hbm.at[j, :, pl.ds(wid, 1)], ssems[b]
        )

    def wait_store(j, b):
        pltpu.make_async_copy(
            tbuf.at[b], out_hbm.at[j, :, pl.ds(wid, 1)], ssems[b]
        ).wait()

    for j in range(_G):
        start_gather(j, j)

    for b in range(_NBUF):
        wait_gather(b, b)
        start_gather(b + _G, (b + _G) % _NBUF)
        transpose(b)
        start_store(b, b)

    def body(k, _):
        i0 = k * _NBUF
        for b in range(_NBUF):
            i = i0 + b
            wait_gather(i, b)
            wait_store(i - _NBUF, b)
            start_gather(i + _G, (b + _G) % _NBUF)
            transpose(b)
            start_store(i, b)
        return ()

    lax.fori_loop(1, _NOUT - 1, body, ())

    i0 = (_NOUT - 1) * _NBUF
    for b in range(_NBUF):
        i = i0 + b
        wait_gather(i, b)
        wait_store(i - _NBUF, b)
        if b < _NBUF - _G:
            start_gather(i + _G, (b + _G) % _NBUF)
        transpose(b)
        start_store(i, b)

    for b in range(_NBUF):
        wait_store(i0 + b, b)


def kernel(token_ids, shard_weights):
    ids_t = token_ids.T
    table = shard_weights.reshape(_NUM_SHARDS * _SHARD_SIZE, _EMBED_DIM)
    out5 = _gather_kernel(ids_t, table)
    out_phys = jnp.transpose(out5, (0, 1, 3, 2, 4)).reshape(
        _SEQ_LEN, _EMBED_DIM, _BATCH
    )
    return jnp.transpose(out_phys, (2, 0, 1))


# speedup vs baseline: 7.1046x; 1.0035x over previous
import functools

import jax
import jax.numpy as jnp
from jax import lax
from jax.experimental import pallas as pl
from jax.experimental.pallas import tpu as pltpu
from jax.experimental.pallas import tpu_sc as plsc

_SHARD_SIZE = 2048
_NUM_SHARDS = 49
_EMBED_DIM = 64
_BATCH = 4096
_SEQ_LEN = 200

_NW = 32
_CHUNK = 128
_NTC = _BATCH // _CHUNK
_NCH = _SEQ_LEN
_NBUF = 5
_G = 4
_NOUT = _NCH // _NBUF

_mesh = plsc.VectorSubcoreMesh(core_axis_name="c", subcore_axis_name="s")


@functools.partial(
    pl.kernel,
    out_type=jax.ShapeDtypeStruct(
        (_SEQ_LEN, _EMBED_DIM // 8, _NTC, 8, _CHUNK), jnp.float32
    ),
    mesh=_mesh,
    compiler_params=pltpu.CompilerParams(
        use_tc_tiling_on_sc=False, needs_layout_passes=False
    ),
    scratch_types=[
        pltpu.VMEM((_SEQ_LEN, _CHUNK), jnp.int32),
        pltpu.VMEM((_NBUF, _CHUNK, _EMBED_DIM), jnp.float32),
        pltpu.VMEM((_NBUF, _EMBED_DIM // 8, 1, 8, _CHUNK), jnp.float32),
        [pltpu.SemaphoreType.DMA] * _NBUF,
        [pltpu.SemaphoreType.DMA] * _NBUF,
    ],
)
def _gather_kernel(ids_hbm, table_hbm, out_hbm, idx_v, rows, tbuf, gsems, ssems):
    wid = lax.axis_index("s") * 2 + lax.axis_index("c")

    pltpu.sync_copy(ids_hbm.at[:, pl.ds(wid * _CHUNK, _CHUNK)], idx_v)

    riota = lax.iota(jnp.int32, 16)
    cvecs = [riota + 16 * k for k in range(8)]
    zvec = jnp.zeros((16,), jnp.int32)

    def start_gather(j, b):
        pltpu.async_copy(table_hbm.at[idx_v.at[j]], rows.at[b], gsems[b])

    def wait_gather(j, b):
        pltpu.make_async_copy(
            table_hbm.at[idx_v.at[j]], rows.at[b], gsems[b]
        ).wait()

    def transpose(b):
        @plsc.parallel_loop(0, _EMBED_DIM, unroll=16)
        def _(d):
            dvec = (riota + d) & 63
            trv = dvec >> 3
            rv = dvec & 7
            for k in range(8):
                v = plsc.load_gather(rows.at[b], [cvecs[k], dvec])
                plsc.store_scatter(
                    tbuf.at[b], [trv, zvec, rv, cvecs[k]], v
                )

    def start_store(j, b):
        pltpu.async_copy(
            tbuf.at[b], out_hbm.at[j, :, pl.ds(wid, 1)], ssems[b]
        )

    def wait_store(j, b):
        pltpu.make_async_copy(
            tbuf.at[b], out_hbm.at[j, :, pl.ds(wid, 1)], ssems[b]
        ).wait()

    for j in range(_G):
        start_gather(j, j)

    for b in range(_NBUF):
        wait_gather(b, b)
        start_gather(b + _G, (b + _G) % _NBUF)
        transpose(b)
        start_store(b, b)

    def body(k, _):
        i0 = k * _NBUF
        for b in range(_NBUF):
            i = i0 + b
            wait_gather(i, b)
            wait_store(i - _NBUF, b)
            start_gather(i + _G, (b + _G) % _NBUF)
            transpose(b)
            start_store(i, b)
        return ()

    lax.fori_loop(1, _NOUT - 1, body, ())

    i0 = (_NOUT - 1) * _NBUF
    for b in range(_NBUF):
        i = i0 + b
        wait_gather(i, b)
        wait_store(i - _NBUF, b)
        if b < _NBUF - _G:
            start_gather(i + _G, (b + _G) % _NBUF)
        transpose(b)
        start_store(i, b)

    for b in range(_NBUF):
        wait_store(i0 + b, b)


def kernel(token_ids, shard_weights):
    ids_t = token_ids.T
    table = shard_weights.reshape(_NUM_SHARDS * _SHARD_SIZE, _EMBED_DIM)
    out5 = _gather_kernel(ids_t, table)
    out_phys = jnp.transpose(out5, (0, 1, 3, 2, 4)).reshape(
        _SEQ_LEN, _EMBED_DIM, _BATCH
    )
    return jnp.transpose(out_phys, (2, 0, 1))


# speedup vs baseline: 7.3265x; 1.0312x over previous
import functools

import jax
import jax.numpy as jnp
from jax import lax
from jax.experimental import pallas as pl
from jax.experimental.pallas import tpu as pltpu
from jax.experimental.pallas import tpu_sc as plsc

_SHARD_SIZE = 2048
_NUM_SHARDS = 49
_EMBED_DIM = 64
_BATCH = 4096
_SEQ_LEN = 200
_VROWS = _NUM_SHARDS * _SHARD_SIZE

_NW = 32
_CHUNK = 128
_NTC = _BATCH // _CHUNK
_NCH = _SEQ_LEN
_NBUF = 4
_G = 3
_NOUT = _NCH // _NBUF

_NUNIT = _NUM_SHARDS * 16
_UPW = _NUNIT // 16

_mesh = plsc.VectorSubcoreMesh(core_axis_name="c", subcore_axis_name="s")


@functools.partial(
    pl.kernel,
    out_type=(
        jax.ShapeDtypeStruct(
            (_SEQ_LEN, _EMBED_DIM // 8, _NTC, 8, _CHUNK), jnp.float32
        ),
        jax.ShapeDtypeStruct((2 * _VROWS, _EMBED_DIM), jnp.float32),
    ),
    mesh=_mesh,
    compiler_params=pltpu.CompilerParams(
        use_tc_tiling_on_sc=False, needs_layout_passes=False
    ),
    scratch_types=[
        pltpu.VMEM((_SEQ_LEN, _CHUNK), jnp.int32),
        pltpu.VMEM((_NBUF, _CHUNK, _EMBED_DIM), jnp.float32),
        pltpu.VMEM((_NBUF, _EMBED_DIM // 8, 1, 8, _CHUNK), jnp.float32),
        pltpu.VMEM((2, 8, 1, 8, 128), jnp.float32),
        pltpu.VMEM((2, 128, _EMBED_DIM), jnp.float32),
        [pltpu.SemaphoreType.DMA] * _NBUF,
        [pltpu.SemaphoreType.DMA] * _NBUF,
        [pltpu.SemaphoreType.DMA] * 2,
        [pltpu.SemaphoreType.DMA] * 2,
    ],
)
def _gather_kernel(
    ids_hbm, w5_hbm, out_hbm, tab_hbm,
    idx_v, rows, tbuf, ttile, trg, gsems, ssems, tisems, tosems,
):
    cid = lax.axis_index("c")
    sid = lax.axis_index("s")
    wid = sid * 2 + cid
    tab_base = cid * _VROWS

    pltpu.sync_copy(ids_hbm.at[:, pl.ds(wid * _CHUNK, _CHUNK)], idx_v)

    riota = lax.iota(jnp.int32, 16)
    cvecs = [riota + 16 * k for k in range(8)]
    zvec = jnp.zeros((16,), jnp.int32)

    @plsc.parallel_loop(0, _SEQ_LEN, unroll=2)
    def _(j):
        for k in range(8):
            idx_v[j, pl.ds(16 * k, 16)] = (
                idx_v[j, pl.ds(16 * k, 16)] + tab_base
            )

    def unit_coords(u):
        return lax.div(u, 16), lax.rem(u, 16)

    def fmt_start_in(u, p):
        sh, otc = unit_coords(u)
        pltpu.async_copy(
            w5_hbm.at[sh, :, pl.ds(otc, 1)], ttile.at[p], tisems[p]
        )

    def fmt_wait_in(u, p):
        sh, otc = unit_coords(u)
        pltpu.make_async_copy(
            w5_hbm.at[sh, :, pl.ds(otc, 1)], ttile.at[p], tisems[p]
        ).wait()

    def fmt_rows(u):
        sh, otc = unit_coords(u)
        return tab_base + (sh * 16 + otc) * 128

    def fmt_start_out(u, p):
        pltpu.async_copy(
            trg.at[p], tab_hbm.at[pl.ds(fmt_rows(u), 128)], tosems[p]
        )

    def fmt_wait_out(u, p):
        pltpu.make_async_copy(
            trg.at[p], tab_hbm.at[pl.ds(fmt_rows(u), 128)], tosems[p]
        ).wait()

    def fmt_transpose(p):
        @plsc.parallel_loop(0, _EMBED_DIM, unroll=8)
        def _(dbase):
            dvec = (riota + dbase) & 63
            etrv = dvec >> 3
            erv = dvec & 7
            for k in range(8):
                v = plsc.load_gather(ttile.at[p], [etrv, zvec, erv, cvecs[k]])
                plsc.store_scatter(trg.at[p], [cvecs[k], dvec], v)

    def unit_id(j):
        return sid * _UPW + j

    fmt_start_in(unit_id(0), 0)
    fmt_start_in(unit_id(1), 1)

    for par in range(2):
        u = unit_id(par)
        fmt_wait_in(u, par)
        fmt_transpose(par)
        fmt_start_out(u, par)
        fmt_start_in(unit_id(par + 2), par)

    def fmt_body(m, _):
        for par in range(2):
            j = 2 * m + par
            u = unit_id(j)

            @pl.when(j < _UPW)
            def _():
                fmt_wait_in(u, par)
                fmt_wait_out(unit_id(j - 2), par)
                fmt_transpose(par)
                fmt_start_out(u, par)

                @pl.when(j + 2 < _UPW)
                def _():
                    fmt_start_in(unit_id(j + 2), par)
        return ()

    lax.fori_loop(1, (_UPW + 1) // 2 + 1, fmt_body, ())

    fmt_wait_out(unit_id(_UPW - 2), (_UPW - 2) % 2)
    fmt_wait_out(unit_id(_UPW - 1), (_UPW - 1) % 2)
    plsc.subcore_barrier()

    def start_gather(j, b):
        pltpu.async_copy(tab_hbm.at[idx_v.at[j]], rows.at[b], gsems[b])

    def wait_gather(j, b):
        pltpu.make_async_copy(
            tab_hbm.at[idx_v.at[j]], rows.at[b], gsems[b]
        ).wait()

    def transpose(b):
        @plsc.parallel_loop(0, _EMBED_DIM, unroll=8)
        def _(d):
            dvec = (riota + d) & 63
            trv = dvec >> 3
            rv = dvec & 7
            for k in range(8):
                v = plsc.load_gather(rows.at[b], [cvecs[k], dvec])
                plsc.store_scatter(tbuf.at[b], [trv, zvec, rv, cvecs[k]], v)

    def start_store(j, b):
        pltpu.async_copy(
            tbuf.at[b], out_hbm.at[j, :, pl.ds(wid, 1)], ssems[b]
        )

    def wait_store(j, b):
        pltpu.make_async_copy(
            tbuf.at[b], out_hbm.at[j, :, pl.ds(wid, 1)], ssems[b]
        ).wait()

    for j in range(_G):
        start_gather(j, j)

    for b in range(_NBUF):
        wait_gather(b, b)
        start_gather(b + _G, (b + _G) % _NBUF)
        transpose(b)
        start_store(b, b)

    def body(k, _):
        i0 = k * _NBUF
        for b in range(_NBUF):
            i = i0 + b
            wait_gather(i, b)
            wait_store(i - _NBUF, b)
            start_gather(i + _G, (b + _G) % _NBUF)
            transpose(b)
            start_store(i, b)
        return ()

    lax.fori_loop(1, _NOUT - 1, body, ())

    i0 = (_NOUT - 1) * _NBUF
    for b in range(_NBUF):
        i = i0 + b
        wait_gather(i, b)
        wait_store(i - _NBUF, b)
        if b < _NBUF - _G:
            start_gather(i + _G, (b + _G) % _NBUF)
        transpose(b)
        start_store(i, b)

    for b in range(_NBUF):
        wait_store(i0 + b, b)


def kernel(token_ids, shard_weights):
    ids_t = token_ids.T
    w5 = jnp.transpose(
        shard_weights.reshape(_NUM_SHARDS, 16, 128, 8, 8), (0, 3, 1, 4, 2)
    )
    out5, _ = _gather_kernel(ids_t, w5)
    out_phys = jnp.transpose(out5, (0, 1, 3, 2, 4)).reshape(
        _SEQ_LEN, _EMBED_DIM, _BATCH
    )
    return jnp.transpose(out_phys, (2, 0, 1))


# speedup vs baseline: 7.4925x; 1.0227x over previous
import functools

import jax
import jax.numpy as jnp
from jax import lax
from jax.experimental import pallas as pl
from jax.experimental.pallas import tpu as pltpu
from jax.experimental.pallas import tpu_sc as plsc

_SHARD_SIZE = 2048
_NUM_SHARDS = 49
_EMBED_DIM = 64
_BATCH = 4096
_SEQ_LEN = 200
_VROWS = _NUM_SHARDS * _SHARD_SIZE

_NW = 32
_CHUNK = 128
_NTC = _BATCH // _CHUNK
_NCH = _SEQ_LEN
_NBUF = 4
_G = 3
_NOUT = _NCH // _NBUF

_NUNIT = _NUM_SHARDS * 16
_UPW = _NUNIT // 16

_mesh = plsc.VectorSubcoreMesh(core_axis_name="c", subcore_axis_name="s")


@functools.partial(
    pl.kernel,
    out_type=(
        jax.ShapeDtypeStruct(
            (_SEQ_LEN, _EMBED_DIM // 8, _NTC, 8, _CHUNK), jnp.float32
        ),
        jax.ShapeDtypeStruct((2 * _VROWS, _EMBED_DIM), jnp.float32),
    ),
    mesh=_mesh,
    compiler_params=pltpu.CompilerParams(
        use_tc_tiling_on_sc=False, needs_layout_passes=False
    ),
    scratch_types=[
        pltpu.VMEM((_SEQ_LEN, _CHUNK), jnp.int32),
        pltpu.VMEM((_NBUF, _CHUNK, _EMBED_DIM), jnp.float32),
        pltpu.VMEM((_NBUF, _EMBED_DIM // 8, 1, 8, _CHUNK), jnp.float32),
        pltpu.VMEM((2, 8, 1, 8, 128), jnp.float32),
        pltpu.VMEM((2, 128, _EMBED_DIM), jnp.float32),
        [pltpu.SemaphoreType.DMA] * _NBUF,
        [pltpu.SemaphoreType.DMA] * _NBUF,
        [pltpu.SemaphoreType.DMA] * 2,
        [pltpu.SemaphoreType.DMA] * 2,
    ],
)
def _gather_kernel(
    ids_hbm, w5_hbm, out_hbm, tab_hbm,
    idx_v, rows, tbuf, ttile, trg, gsems, ssems, tisems, tosems,
):
    cid = lax.axis_index("c")
    sid = lax.axis_index("s")
    wid = sid * 2 + cid
    tab_base = cid * _VROWS

    pltpu.sync_copy(ids_hbm.at[:, pl.ds(wid * _CHUNK, _CHUNK)], idx_v)

    riota = lax.iota(jnp.int32, 16)
    cvecs = [riota + 16 * k for k in range(8)]
    zvec = jnp.zeros((16,), jnp.int32)

    @plsc.parallel_loop(0, _SEQ_LEN, unroll=2)
    def _(j):
        for k in range(8):
            idx_v[j, pl.ds(16 * k, 16)] = (
                idx_v[j, pl.ds(16 * k, 16)] + tab_base
            )

    def unit_coords(u):
        return lax.div(u, 16), lax.rem(u, 16)

    def fmt_start_in(u, p):
        sh, otc = unit_coords(u)
        pltpu.async_copy(
            w5_hbm.at[sh, :, pl.ds(otc, 1)], ttile.at[p], tisems[p]
        )

    def fmt_wait_in(u, p):
        sh, otc = unit_coords(u)
        pltpu.make_async_copy(
            w5_hbm.at[sh, :, pl.ds(otc, 1)], ttile.at[p], tisems[p]
        ).wait()

    def fmt_rows(u):
        sh, otc = unit_coords(u)
        return tab_base + (sh * 16 + otc) * 128

    def fmt_start_out(u, p):
        pltpu.async_copy(
            trg.at[p], tab_hbm.at[pl.ds(fmt_rows(u), 128)], tosems[p]
        )

    def fmt_wait_out(u, p):
        pltpu.make_async_copy(
            trg.at[p], tab_hbm.at[pl.ds(fmt_rows(u), 128)], tosems[p]
        ).wait()

    def fmt_transpose(p):
        @plsc.parallel_loop(0, _EMBED_DIM, unroll=8)
        def _(dbase):
            dvec = (riota + dbase) & 63
            etrv = dvec >> 3
            erv = dvec & 7
            for k in range(8):
                v = plsc.load_gather(ttile.at[p], [etrv, zvec, erv, cvecs[k]])
                plsc.store_scatter(trg.at[p], [cvecs[k], dvec], v)

    def unit_id(j):
        return sid * _UPW + j

    fmt_start_in(unit_id(0), 0)
    fmt_start_in(unit_id(1), 1)

    for par in range(2):
        u = unit_id(par)
        fmt_wait_in(u, par)
        fmt_transpose(par)
        fmt_start_out(u, par)
        fmt_start_in(unit_id(par + 2), par)

    def fmt_body(m, _):
        for par in range(2):
            j = 2 * m + par
            u = unit_id(j)

            @pl.when(j < _UPW)
            def _():
                fmt_wait_in(u, par)
                fmt_wait_out(unit_id(j - 2), par)
                fmt_transpose(par)
                fmt_start_out(u, par)

                @pl.when(j + 2 < _UPW)
                def _():
                    fmt_start_in(unit_id(j + 2), par)
        return ()

    lax.fori_loop(1, (_UPW + 1) // 2 + 1, fmt_body, ())

    fmt_wait_out(unit_id(_UPW - 2), (_UPW - 2) % 2)
    fmt_wait_out(unit_id(_UPW - 1), (_UPW - 1) % 2)
    plsc.subcore_barrier()

    def start_gather(j, b):
        pltpu.async_copy(tab_hbm.at[idx_v.at[j]], rows.at[b], gsems[b])

    def wait_gather(j, b):
        pltpu.make_async_copy(
            tab_hbm.at[idx_v.at[j]], rows.at[b], gsems[b]
        ).wait()

    def transpose(b):
        @plsc.parallel_loop(0, _EMBED_DIM, unroll=16)
        def _(d):
            dvec = (riota + d) & 63
            trv = dvec >> 3
            rv = dvec & 7
            for k in range(8):
                v = plsc.load_gather(rows.at[b], [cvecs[k], dvec])
                plsc.store_scatter(tbuf.at[b], [trv, zvec, rv, cvecs[k]], v)

    def start_store(j, b):
        pltpu.async_copy(
            tbuf.at[b], out_hbm.at[j, :, pl.ds(wid, 1)], ssems[b]
        )

    def wait_store(j, b):
        pltpu.make_async_copy(
            tbuf.at[b], out_hbm.at[j, :, pl.ds(wid, 1)], ssems[b]
        ).wait()

    for j in range(_G):
        start_gather(j, j)

    for b in range(_NBUF):
        wait_gather(b, b)
        start_gather(b + _G, (b + _G) % _NBUF)
        transpose(b)
        start_store(b, b)

    def body(k, _):
        i0 = k * _NBUF
        for b in range(_NBUF):
            i = i0 + b
            wait_gather(i, b)
            wait_store(i - _NBUF, b)
            start_gather(i + _G, (b + _G) % _NBUF)
            transpose(b)
            start_store(i, b)
        return ()

    lax.fori_loop(1, _NOUT - 1, body, ())

    i0 = (_NOUT - 1) * _NBUF
    for b in range(_NBUF):
        i = i0 + b
        wait_gather(i, b)
        wait_store(i - _NBUF, b)
        if b < _NBUF - _G:
            start_gather(i + _G, (b + _G) % _NBUF)
        transpose(b)
        start_store(i, b)

    for b in range(_NBUF):
        wait_store(i0 + b, b)


def kernel(token_ids, shard_weights):
    ids_t = token_ids.T
    w5 = jnp.transpose(
        shard_weights.reshape(_NUM_SHARDS, 16, 128, 8, 8), (0, 3, 1, 4, 2)
    )
    out5, _ = _gather_kernel(ids_t, w5)
    out_phys = jnp.transpose(out5, (0, 1, 3, 2, 4)).reshape(
        _SEQ_LEN, _EMBED_DIM, _BATCH
    )
    return jnp.transpose(out_phys, (2, 0, 1))
